# baseline (speedup 1.0000x reference)
import jax
import jax.numpy as jnp
from jax.experimental import pallas as pl

_SIZE = 32000
_PADDING_IDX = 0
_SMOOTHING = 0.1
_CONFIDENCE = 1.0 - _SMOOTHING
_FILL = _SMOOTHING / (_SIZE - 2)

_ROWS_PER_BLOCK = 64


def _fill_kernel(tgt_ref, out_ref):
    r, c = out_ref.shape
    tgt = tgt_ref[0].reshape(r, 1)
    col = jax.lax.broadcasted_iota(jnp.int32, (r, c), 1)
    vals = jnp.where(col == tgt, _CONFIDENCE, _FILL)
    vals = jnp.where(col == _PADDING_IDX, 0.0, vals)
    vals = jnp.where(tgt == _PADDING_IDX, 0.0, vals)
    out_ref[...] = vals.astype(out_ref.dtype)


def kernel(x, target):
    n, size = x.shape
    assert size == _SIZE
    rb = _ROWS_PER_BLOCK
    num_blocks = n // rb
    tgt = target.astype(jnp.int32).reshape(num_blocks, 1, rb)
    return pl.pallas_call(
        _fill_kernel,
        grid=(num_blocks,),
        in_specs=[pl.BlockSpec((1, 1, rb), lambda i: (i, 0, 0))],
        out_specs=pl.BlockSpec((rb, size), lambda i: (i, 0)),
        out_shape=jax.ShapeDtypeStruct((n, size), x.dtype),
    )(tgt)
